# Initial kernel scaffold; baseline (speedup 1.0000x reference)
import jax
import jax.numpy as jnp
from jax.experimental import pallas as pl


def kernel(emb, nodes, pos_edges, neg_edges):
    raise NotImplementedError("write your pallas kernel here")


# speedup vs baseline: 4.5828x; 4.5828x over previous
import functools

import jax
import jax.numpy as jnp
from jax import lax
from jax.experimental import pallas as pl
from jax.experimental.pallas import tpu as pltpu
from jax.experimental.pallas import tpu_sc as plsc

_DELTA = 1.0
_N_NODES = 50000
_N_PAD = 50048
_E = 200000
_CHUNK = 64
_E_PAD = 200704
_D = 128
_NW = 32
_CHUNKS_PER_TILE = _E_PAD // _CHUNK // _NW
_NEG_INIT = -3.0e38


def _splat_i32(v):
  return jnp.full((16,), v, dtype=jnp.int32)


def _rsqrt16(x):
  y = jnp.full((16,), 0.7071067811865476 * 2.0**20, jnp.float32)
  for k in range(-20, 9):
    y = jnp.where(x >= 4.0**k, jnp.float32(0.7071067811865476 * 2.0**-k), y)
  for _ in range(4):
    y = y * (1.5 - 0.5 * x * y * y)
  return y


def _hsum16(v, lane):
  for sh in (1, 2, 4, 8):
    v = v + v.at[lane ^ sh].get(mode="promise_in_bounds")
  return v


def _edge_distances(rows_a, rows_b, e0):
  lane = lax.iota(jnp.int32, 16)
  sums = jnp.zeros((16,), jnp.float32)
  for j in range(16):
    e = e0 + j
    acc = jnp.zeros((16,), jnp.float32)
    for blk in range(8):
      a = rows_a[e, pl.ds(blk * 16, 16)]
      b = rows_b[e, pl.ds(blk * 16, 16)]
      t = a - b
      acc = acc + t * t
    sums = jnp.where(lane == j, _hsum16(acc, lane), sums)
  x = sums + 1e-12
  return -(x * _rsqrt16(x))


def _sc_body(emb, nodes, pos_seg, pos_sg, pos_dg, neg_seg, neg_sg, neg_dg,
             pos_sum_out, pos_cnt_out, neg_max_out,
             seg_v, gs_v, gd_v, nid_s, nid_d, rows_a, rows_b,
             vals_v, ones_v, big_v, sp_sum, sp_cnt, sem):
  cid = lax.axis_index("c")
  sid = lax.axis_index("s")
  wid = sid * 2 + cid

  for i in range(4):
    ones_v[pl.ds(i * 16, 16)] = jnp.full((16,), 1.0, jnp.float32)

  def zero_blk(i, _):
    big_v[pl.ds(i * 16, 16)] = jnp.zeros((16,), jnp.float32)
    return 0

  lax.fori_loop(0, _N_PAD // 16, zero_blk, 0)

  @pl.when(sid == 0)
  def _():
    pltpu.sync_copy(big_v, sp_sum)
    pltpu.sync_copy(big_v, sp_cnt)

  plsc.subcore_barrier()

  def fetch_chunk(chunk, seg_hbm, sg_hbm, dg_hbm):
    base = chunk * _CHUNK
    pltpu.sync_copy(seg_hbm.at[pl.ds(base, _CHUNK)], seg_v)
    pltpu.sync_copy(sg_hbm.at[pl.ds(base, _CHUNK)], gs_v)
    pltpu.sync_copy(dg_hbm.at[pl.ds(base, _CHUNK)], gd_v)
    pltpu.async_copy(nodes.at[gs_v], nid_s, sem).wait()
    pltpu.async_copy(nodes.at[gd_v], nid_d, sem).wait()
    pltpu.async_copy(emb.at[nid_s], rows_a, sem).wait()
    pltpu.async_copy(emb.at[nid_d], rows_b, sem).wait()

  def pos_chunk(k, _):
    chunk = k * _NW + wid
    fetch_chunk(chunk, pos_seg, pos_sg, pos_dg)
    for sub in range(4):
      dval = _edge_distances(rows_a, rows_b, sub * 16)
      vals_v[pl.ds(sub * 16, 16)] = dval
    pltpu.sync_copy(vals_v, sp_sum.at[seg_v], add=True)
    pltpu.sync_copy(ones_v, sp_cnt.at[seg_v], add=True)
    return 0

  lax.fori_loop(0, _CHUNKS_PER_TILE, pos_chunk, 0)
  plsc.subcore_barrier()

  @pl.when(sid == 0)
  def _():
    pltpu.sync_copy(sp_sum, pos_sum_out.at[cid])
    pltpu.sync_copy(sp_cnt, pos_cnt_out.at[cid])

  def neg_init_blk(i, _):
    big_v[pl.ds(i * 16, 16)] = jnp.full((16,), _NEG_INIT, jnp.float32)
    return 0

  lax.fori_loop(0, _N_PAD // 16, neg_init_blk, 0)

  def neg_chunk(k, _):
    chunk = k * _NW + wid
    fetch_chunk(chunk, neg_seg, neg_sg, neg_dg)
    for sub in range(4):
      dval = _edge_distances(rows_a, rows_b, sub * 16)
      seg16 = seg_v[pl.ds(sub * 16, 16)]

      def retry(_, pending):
        cur = plsc.load_gather(big_v, [seg16])
        new = jnp.maximum(cur, dval)
        plsc.store_scatter(big_v, [seg16], new, mask=pending)
        chk = plsc.load_gather(big_v, [seg16])
        return pending & (chk < dval)

      lax.fori_loop(0, 16, retry, jnp.full((16,), True))
    return 0

  lax.fori_loop(0, _CHUNKS_PER_TILE, neg_chunk, 0)
  pltpu.sync_copy(big_v, neg_max_out.at[wid])


def _sc_partials(emb, nodes, pos_seg, pos_sg, pos_dg, neg_seg, neg_sg, neg_dg):
  mesh = plsc.VectorSubcoreMesh(core_axis_name="c", subcore_axis_name="s")
  f32 = jnp.float32
  return pl.kernel(
      _sc_body,
      mesh=mesh,
      compiler_params=pltpu.CompilerParams(needs_layout_passes=False),
      out_type=[
          jax.ShapeDtypeStruct((2, _N_PAD), f32),
          jax.ShapeDtypeStruct((2, _N_PAD), f32),
          jax.ShapeDtypeStruct((_NW, _N_PAD), f32),
      ],
      scratch_types=[
          pltpu.VMEM((_CHUNK,), jnp.int32),
          pltpu.VMEM((_CHUNK,), jnp.int32),
          pltpu.VMEM((_CHUNK,), jnp.int32),
          pltpu.VMEM((_CHUNK,), jnp.int32),
          pltpu.VMEM((_CHUNK,), jnp.int32),
          pltpu.VMEM((_CHUNK, _D), f32),
          pltpu.VMEM((_CHUNK, _D), f32),
          pltpu.VMEM((_CHUNK,), f32),
          pltpu.VMEM((_CHUNK,), f32),
          pltpu.VMEM((_N_PAD,), f32),
          pltpu.VMEM_SHARED((_N_PAD,), f32),
          pltpu.VMEM_SHARED((_N_PAD,), f32),
          pltpu.SemaphoreType.DMA,
      ],
  )(emb, nodes, pos_seg, pos_sg, pos_dg, neg_seg, neg_sg, neg_dg)


def _merge_body(ps_ref, pc_ref, nm_ref, out_ref):
  s = jnp.sum(ps_ref[...], axis=0, keepdims=True)
  c = jnp.sum(pc_ref[...], axis=0, keepdims=True)
  p_d = s / jnp.maximum(c, 1.0)
  m = jnp.max(nm_ref[...], axis=0, keepdims=True)
  n_d = jnp.where(m < -1.0e37, 0.0, m)
  hinge = jnp.maximum(n_d - p_d + _DELTA, 0.0)
  valid = lax.broadcasted_iota(jnp.int32, (1, _N_PAD), 1) < _N_NODES
  total = jnp.sum(jnp.where(valid, hinge, 0.0)) / float(_N_NODES)
  out_ref[...] = jnp.broadcast_to(total, (1, 1))


def _merge(pos_sum_p, pos_cnt_p, neg_max_p):
  return pl.pallas_call(
      _merge_body,
      out_shape=jax.ShapeDtypeStruct((1, 1), jnp.float32),
  )(pos_sum_p, pos_cnt_p, neg_max_p)


@jax.jit
def kernel(emb, nodes, pos_edges, neg_edges):
  npad = _E_PAD - _E
  zeros = jnp.zeros((npad,), jnp.int32)
  segpad = jnp.full((npad,), _N_NODES, jnp.int32)

  def split(edges):
    seg = jnp.concatenate([edges[0], segpad])
    sg = jnp.concatenate([edges[0], zeros])
    dg = jnp.concatenate([edges[1], zeros])
    return seg, sg, dg

  pos_seg, pos_sg, pos_dg = split(pos_edges)
  neg_seg, neg_sg, neg_dg = split(neg_edges)
  ps, pc, nm = _sc_partials(emb, nodes, pos_seg, pos_sg, pos_dg,
                            neg_seg, neg_sg, neg_dg)
  return _merge(ps, pc, nm)[0, 0]


# speedup vs baseline: 8.3527x; 1.8226x over previous
import functools

import jax
import jax.numpy as jnp
from jax import lax
from jax.experimental import pallas as pl
from jax.experimental.pallas import tpu as pltpu
from jax.experimental.pallas import tpu_sc as plsc

_DELTA = 1.0
_N_NODES = 50000
_N_PAD = 50048
_E = 200000
_CHUNK = 64
_E_PAD = 200704
_D = 128
_NW = 32
_CHUNKS_PER_TILE = _E_PAD // _CHUNK // _NW
_NEG_INIT = -3.0e38


def _splat_i32(v):
  return jnp.full((16,), v, dtype=jnp.int32)


def _rsqrt16(x):
  y = jnp.full((16,), 0.7071067811865476 * 2.0**20, jnp.float32)
  for k in range(-20, 9):
    y = jnp.where(x >= 4.0**k, jnp.float32(0.7071067811865476 * 2.0**-k), y)
  for _ in range(4):
    y = y * (1.5 - 0.5 * x * y * y)
  return y


def _hsum16(v, lane):
  for sh in (1, 2, 4, 8):
    v = v + v.at[lane ^ sh].get(mode="promise_in_bounds")
  return v


def _edge_distances(rows_a, rows_b, e0):
  lane = lax.iota(jnp.int32, 16)
  sums = jnp.zeros((16,), jnp.float32)
  for j in range(16):
    e = e0 + j

    def dblk(blk, acc, e=e):
      a = rows_a[e, pl.ds(blk * 16, 16)]
      b = rows_b[e, pl.ds(blk * 16, 16)]
      t = a - b
      return acc + t * t

    acc = lax.fori_loop(0, 8, dblk, jnp.zeros((16,), jnp.float32))
    sums = jnp.where(lane == j, _hsum16(acc, lane), sums)
  x = sums + 1e-12
  return -(x * _rsqrt16(x))


def _sc_body(emb, nodes, pos_seg, pos_sg, pos_dg, neg_seg, neg_sg, neg_dg,
             pos_sum_out, pos_cnt_out, neg_max_out,
             seg_a, gs_a, gd_a, nid_sa, nid_da, rows_aa, rows_ba,
             seg_b, gs_b, gd_b, nid_sb, nid_db, rows_ab, rows_bb,
             vals_v, ones_v, big_v, sp_sum, sp_cnt, sem, sem_a, sem_b):
  cid = lax.axis_index("c")
  sid = lax.axis_index("s")
  wid = sid * 2 + cid

  for i in range(4):
    ones_v[pl.ds(i * 16, 16)] = jnp.full((16,), 1.0, jnp.float32)

  def zero_blk(i, _):
    big_v[pl.ds(i * 16, 16)] = jnp.zeros((16,), jnp.float32)
    return 0

  lax.fori_loop(0, _N_PAD // 16, zero_blk, 0)

  @pl.when(sid == 0)
  def _():
    pltpu.sync_copy(big_v, sp_sum)
    pltpu.sync_copy(big_v, sp_cnt)

  plsc.subcore_barrier()

  bufs = (
      (seg_a, gs_a, gd_a, nid_sa, nid_da, rows_aa, rows_ba, sem_a),
      (seg_b, gs_b, gd_b, nid_sb, nid_db, rows_ab, rows_bb, sem_b),
  )

  def fetch_pair(t, seg_hbm, sg_hbm, dg_hbm):
    hs = []
    for p in range(2):
      seg_v, gs_v, gd_v = bufs[p][0], bufs[p][1], bufs[p][2]
      base = ((2 * t + p) * _NW + wid) * _CHUNK
      hs.append(pltpu.async_copy(seg_hbm.at[pl.ds(base, _CHUNK)], seg_v, sem))
      hs.append(pltpu.async_copy(sg_hbm.at[pl.ds(base, _CHUNK)], gs_v, sem))
      hs.append(pltpu.async_copy(dg_hbm.at[pl.ds(base, _CHUNK)], gd_v, sem))
    for h in hs:
      h.wait()
    hs = []
    for p in range(2):
      gs_v, gd_v, nid_s, nid_d = bufs[p][1], bufs[p][2], bufs[p][3], bufs[p][4]
      hs.append(pltpu.async_copy(nodes.at[gs_v], nid_s, sem))
      hs.append(pltpu.async_copy(nodes.at[gd_v], nid_d, sem))
    for h in hs:
      h.wait()
    rhs = []
    for p in range(2):
      nid_s, nid_d, rows_a, rows_b, sem_p = (bufs[p][3], bufs[p][4],
                                             bufs[p][5], bufs[p][6], bufs[p][7])
      rhs.append(pltpu.async_copy(emb.at[nid_s], rows_a, sem_p))
      rhs.append(pltpu.async_copy(emb.at[nid_d], rows_b, sem_p))
    return rhs

  def pos_pair(t, _):
    rhs = fetch_pair(t, pos_seg, pos_sg, pos_dg)
    for p in range(2):
      seg_v, rows_a, rows_b = bufs[p][0], bufs[p][5], bufs[p][6]
      rhs[2 * p].wait()
      rhs[2 * p + 1].wait()

      def pos_sub(sub, _, rows_a=rows_a, rows_b=rows_b):
        dval = _edge_distances(rows_a, rows_b, sub * 16)
        vals_v[pl.ds(sub * 16, 16)] = dval
        return 0

      lax.fori_loop(0, 4, pos_sub, 0)
      pltpu.sync_copy(vals_v, sp_sum.at[seg_v], add=True)
      pltpu.sync_copy(ones_v, sp_cnt.at[seg_v], add=True)
    return 0

  lax.fori_loop(0, _CHUNKS_PER_TILE // 2, pos_pair, 0)
  plsc.subcore_barrier()

  @pl.when(sid == 0)
  def _():
    pltpu.sync_copy(sp_sum, pos_sum_out.at[cid])
    pltpu.sync_copy(sp_cnt, pos_cnt_out.at[cid])

  def neg_init_blk(i, _):
    big_v[pl.ds(i * 16, 16)] = jnp.full((16,), _NEG_INIT, jnp.float32)
    return 0

  lax.fori_loop(0, _N_PAD // 16, neg_init_blk, 0)

  def neg_pair(t, _):
    rhs = fetch_pair(t, neg_seg, neg_sg, neg_dg)
    for p in range(2):
      seg_v, rows_a, rows_b = bufs[p][0], bufs[p][5], bufs[p][6]
      rhs[2 * p].wait()
      rhs[2 * p + 1].wait()

      def neg_sub(sub, _, seg_v=seg_v, rows_a=rows_a, rows_b=rows_b):
        dval = _edge_distances(rows_a, rows_b, sub * 16)
        seg16 = seg_v[pl.ds(sub * 16, 16)]

        def retry(_, pending):
          cur = plsc.load_gather(big_v, [seg16])
          new = jnp.maximum(cur, dval)
          plsc.store_scatter(big_v, [seg16], new, mask=pending)
          chk = plsc.load_gather(big_v, [seg16])
          return pending & (chk < dval)

        lax.fori_loop(0, 16, retry, jnp.full((16,), True))
        return 0

      lax.fori_loop(0, 4, neg_sub, 0)
    return 0

  lax.fori_loop(0, _CHUNKS_PER_TILE // 2, neg_pair, 0)
  pltpu.sync_copy(big_v, neg_max_out.at[wid])


def _sc_partials(emb, nodes, pos_seg, pos_sg, pos_dg, neg_seg, neg_sg, neg_dg):
  mesh = plsc.VectorSubcoreMesh(core_axis_name="c", subcore_axis_name="s")
  f32 = jnp.float32
  return pl.kernel(
      _sc_body,
      mesh=mesh,
      compiler_params=pltpu.CompilerParams(needs_layout_passes=False),
      out_type=[
          jax.ShapeDtypeStruct((2, _N_PAD), f32),
          jax.ShapeDtypeStruct((2, _N_PAD), f32),
          jax.ShapeDtypeStruct((_NW, _N_PAD), f32),
      ],
      scratch_types=(
          [
              pltpu.VMEM((_CHUNK,), jnp.int32),
              pltpu.VMEM((_CHUNK,), jnp.int32),
              pltpu.VMEM((_CHUNK,), jnp.int32),
              pltpu.VMEM((_CHUNK,), jnp.int32),
              pltpu.VMEM((_CHUNK,), jnp.int32),
              pltpu.VMEM((_CHUNK, _D), f32),
              pltpu.VMEM((_CHUNK, _D), f32),
          ] * 2
          + [
              pltpu.VMEM((_CHUNK,), f32),
              pltpu.VMEM((_CHUNK,), f32),
              pltpu.VMEM((_N_PAD,), f32),
              pltpu.VMEM_SHARED((_N_PAD,), f32),
              pltpu.VMEM_SHARED((_N_PAD,), f32),
              pltpu.SemaphoreType.DMA,
              pltpu.SemaphoreType.DMA,
              pltpu.SemaphoreType.DMA,
          ]
      ),
  )(emb, nodes, pos_seg, pos_sg, pos_dg, neg_seg, neg_sg, neg_dg)


def _merge_body(ps_ref, pc_ref, nm_ref, out_ref):
  s = jnp.sum(ps_ref[...], axis=0, keepdims=True)
  c = jnp.sum(pc_ref[...], axis=0, keepdims=True)
  p_d = s / jnp.maximum(c, 1.0)
  m = jnp.max(nm_ref[...], axis=0, keepdims=True)
  n_d = jnp.where(m < -1.0e37, 0.0, m)
  hinge = jnp.maximum(n_d - p_d + _DELTA, 0.0)
  valid = lax.broadcasted_iota(jnp.int32, (1, _N_PAD), 1) < _N_NODES
  total = jnp.sum(jnp.where(valid, hinge, 0.0)) / float(_N_NODES)
  out_ref[...] = jnp.broadcast_to(total, (1, 1))


def _merge(pos_sum_p, pos_cnt_p, neg_max_p):
  return pl.pallas_call(
      _merge_body,
      out_shape=jax.ShapeDtypeStruct((1, 1), jnp.float32),
  )(pos_sum_p, pos_cnt_p, neg_max_p)


@jax.jit
def kernel(emb, nodes, pos_edges, neg_edges):
  npad = _E_PAD - _E
  zeros = jnp.zeros((npad,), jnp.int32)
  segpad = jnp.full((npad,), _N_NODES, jnp.int32)

  def split(edges):
    seg = jnp.concatenate([edges[0], segpad])
    sg = jnp.concatenate([edges[0], zeros])
    dg = jnp.concatenate([edges[1], zeros])
    return seg, sg, dg

  pos_seg, pos_sg, pos_dg = split(pos_edges)
  neg_seg, neg_sg, neg_dg = split(neg_edges)
  ps, pc, nm = _sc_partials(emb, nodes, pos_seg, pos_sg, pos_dg,
                            neg_seg, neg_sg, neg_dg)
  return _merge(ps, pc, nm)[0, 0]
